# baseline (speedup 1.0000x reference)
import functools

import jax
import jax.numpy as jnp
from jax import lax
from jax.experimental import pallas as pl
from jax.experimental.pallas import tpu as pltpu
from jax.experimental.pallas import tpu_sc as plsc

B = 4096
D = 40
NC, NS = 2, 16
NW = NC * NS
BPW = B // NW

H1, H2, H3 = 1024, 512, 256
BS = 512


def _gather_body(pid, uid, dow, mon, hr,
                 ptab, utab, dwtab, motab, hrtab,
                 pe, ue, de, me, he,
                 idx_v, row_v, sem):
    wid = lax.axis_index("s") * NC + lax.axis_index("c")
    base = wid * BPW
    for idx_hbm, tab, out in ((pid, ptab, pe), (uid, utab, ue),
                              (dow, dwtab, de), (mon, motab, me),
                              (hr, hrtab, he)):
        pltpu.sync_copy(idx_hbm.at[pl.ds(base, BPW)], idx_v)
        pltpu.async_copy(tab.at[idx_v], row_v, sem).wait()
        pltpu.sync_copy(row_v, out.at[pl.ds(base, BPW)])


@functools.cache
def _make_gather():
    return pl.kernel(
        _gather_body,
        out_type=[jax.ShapeDtypeStruct((B, D), jnp.float32)] * 5,
        mesh=plsc.VectorSubcoreMesh(core_axis_name="c", subcore_axis_name="s"),
        scratch_types=[
            pltpu.VMEM((BPW,), jnp.int32),
            pltpu.VMEM((BPW, D), jnp.float32),
            pltpu.SemaphoreType.DMA,
        ],
        compiler_params=pltpu.CompilerParams(use_tc_tiling_on_sc=False),
    )


def _mlp_body(pe, ue, de, me, he, rate, pidf, uidf,
              wwide, bwide, w1, b1, w2, b2, w3, b3, out):
    h = (jnp.dot(pe[...], w1[0:D, :], preferred_element_type=jnp.float32)
         + jnp.dot(ue[...], w1[D:2 * D, :], preferred_element_type=jnp.float32)
         + jnp.dot(de[...], w1[2 * D:3 * D, :], preferred_element_type=jnp.float32)
         + jnp.dot(me[...], w1[3 * D:4 * D, :], preferred_element_type=jnp.float32)
         + jnp.dot(he[...], w1[4 * D:5 * D, :], preferred_element_type=jnp.float32)
         + rate[...] * w1[5 * D:5 * D + 1, :]
         + b1[...])
    h = jnp.maximum(h, 0.0)
    h = jnp.maximum(jnp.dot(h, w2[...], preferred_element_type=jnp.float32)
                    + b2[...], 0.0)
    logits = jnp.dot(h, w3[...], preferred_element_type=jnp.float32) + b3[...]
    m = jnp.max(logits, axis=-1, keepdims=True)
    e = jnp.exp(logits - m)
    sm = e / jnp.sum(e, axis=-1, keepdims=True)
    bf = lambda v: v.astype(jnp.bfloat16).astype(jnp.float32)
    wide = (bf(pidf[...]) * bf(wwide[0, 0]) + bf(uidf[...]) * bf(wwide[1, 0])
            + bwide[0, 0])
    z = sm + wide
    out[...] = 1.0 / (1.0 + jnp.exp(-z))


def _make_mlp(interpret=False):
    bspec_rows = lambda w: pl.BlockSpec((BS, w), lambda i: (i, 0))
    bspec_full = lambda r, c: pl.BlockSpec((r, c), lambda i: (0, 0))
    return pl.pallas_call(
        _mlp_body,
        grid=(B // BS,),
        in_specs=[
            bspec_rows(D), bspec_rows(D), bspec_rows(D), bspec_rows(D),
            bspec_rows(D), bspec_rows(1), bspec_rows(1), bspec_rows(1),
            bspec_full(2, 1), bspec_full(1, 1),
            bspec_full(5 * D + 1, H1), bspec_full(1, H1),
            bspec_full(H1, H2), bspec_full(1, H2),
            bspec_full(H2, H3), bspec_full(1, H3),
        ],
        out_specs=bspec_rows(H3),
        out_shape=jax.ShapeDtypeStruct((B, H3), jnp.float32),
        compiler_params=pltpu.CompilerParams(
            dimension_semantics=("arbitrary",)),
        interpret=interpret,
    )


_mlp = _make_mlp()


def kernel(product_id, user_id, day_of_week, month, hour, rate,
           product_table, user_table, dw_table, month_table, hour_table,
           W_wide, b_wide, W1, b1, W2, b2, W3, b3):
    pid32 = product_id.astype(jnp.int32)
    uid32 = user_id.astype(jnp.int32)
    dow32 = day_of_week.astype(jnp.int32)
    mon32 = month.astype(jnp.int32)
    hr32 = hour.astype(jnp.int32)
    pe, ue, de, me, he = _make_gather()(pid32, uid32, dow32, mon32, hr32,
                                        product_table, user_table, dw_table,
                                        month_table, hour_table)
    pidf = pid32.astype(jnp.float32).reshape(B, 1)
    uidf = uid32.astype(jnp.float32).reshape(B, 1)
    return _mlp(pe, ue, de, me, he, rate, pidf, uidf,
                W_wide, b_wide.reshape(1, 1),
                W1, b1.reshape(1, H1), W2, b2.reshape(1, H2),
                W3, b3.reshape(1, H3))


# speedup vs baseline: 2.8252x; 2.8252x over previous
import functools

import jax
import jax.numpy as jnp
from jax import lax
from jax.experimental import pallas as pl
from jax.experimental.pallas import tpu as pltpu
from jax.experimental.pallas import tpu_sc as plsc

B = 4096
D = 40
NC, NS = 2, 16
NW = NC * NS
BPW = B // NW

H1, H2, H3 = 1024, 512, 256
ND, NM, NH = 7, 12, 24
BS = 512


def _gather_body(pid, uid, ptab, utab, pe, ue, idx_v, pe_v, ue_v, sem):
    wid = lax.axis_index("s") * NC + lax.axis_index("c")
    base = wid * BPW
    lanes = lax.iota(jnp.int32, 16)
    handles = []

    def fire(idx_hbm, tab, row_buf):
        pltpu.sync_copy(idx_hbm.at[pl.ds(base, BPW)], idx_v)
        for c in range(BPW // 16):
            chunk = idx_v[pl.ds(c * 16, 16)]
            for l in range(16):
                g = jnp.sum(jnp.where(lanes == l, chunk, 0))
                handles.append(pltpu.async_copy(
                    tab.at[pl.ds(g, 1), :],
                    row_buf.at[pl.ds(c * 16 + l, 1), :], sem))

    fire(pid, ptab, pe_v)
    fire(uid, utab, ue_v)
    for h in handles:
        h.wait()
    pltpu.sync_copy(pe_v, pe.at[pl.ds(base, BPW)])
    pltpu.sync_copy(ue_v, ue.at[pl.ds(base, BPW)])


@functools.cache
def _make_gather():
    return pl.kernel(
        _gather_body,
        out_type=[jax.ShapeDtypeStruct((B, D), jnp.float32)] * 2,
        mesh=plsc.VectorSubcoreMesh(core_axis_name="c", subcore_axis_name="s"),
        scratch_types=[
            pltpu.VMEM((BPW,), jnp.int32),
            pltpu.VMEM((BPW, D), jnp.float32),
            pltpu.VMEM((BPW, D), jnp.float32),
            pltpu.SemaphoreType.DMA,
        ],
        compiler_params=pltpu.CompilerParams(needs_layout_passes=False),
    )


def _mlp_body(pe, ue, dow, mon, hr, rate, pidf, uidf,
              dwtab, motab, hrtab, wwide, bwide,
              w1, b1, w2, b2, w3, b3, out):
    def onehot_rows(ids, n, tab):
        oh = (lax.broadcasted_iota(jnp.int32, (BS, n), 1)
              == ids).astype(jnp.float32)
        return jnp.dot(oh, tab[...], precision=lax.Precision.HIGHEST,
                       preferred_element_type=jnp.float32)

    de = onehot_rows(dow[...], ND, dwtab)
    me = onehot_rows(mon[...], NM, motab)
    he = onehot_rows(hr[...], NH, hrtab)
    h = (jnp.dot(pe[...], w1[0:D, :], preferred_element_type=jnp.float32)
         + jnp.dot(ue[...], w1[D:2 * D, :], preferred_element_type=jnp.float32)
         + jnp.dot(de, w1[2 * D:3 * D, :], preferred_element_type=jnp.float32)
         + jnp.dot(me, w1[3 * D:4 * D, :], preferred_element_type=jnp.float32)
         + jnp.dot(he, w1[4 * D:5 * D, :], preferred_element_type=jnp.float32)
         + rate[...] * w1[5 * D:5 * D + 1, :]
         + b1[...])
    h = jnp.maximum(h, 0.0)
    h = jnp.maximum(jnp.dot(h, w2[...], preferred_element_type=jnp.float32)
                    + b2[...], 0.0)
    logits = jnp.dot(h, w3[...], preferred_element_type=jnp.float32) + b3[...]
    m = jnp.max(logits, axis=-1, keepdims=True)
    e = jnp.exp(logits - m)
    sm = e / jnp.sum(e, axis=-1, keepdims=True)
    bf = lambda v: v.astype(jnp.bfloat16).astype(jnp.float32)
    wide = (bf(pidf[...]) * bf(wwide[0, 0]) + bf(uidf[...]) * bf(wwide[1, 0])
            + bwide[0, 0])
    z = sm + wide
    out[...] = 1.0 / (1.0 + jnp.exp(-z))


def _make_mlp(interpret=False):
    bspec_rows = lambda w: pl.BlockSpec((BS, w), lambda i: (i, 0))
    bspec_full = lambda r, c: pl.BlockSpec((r, c), lambda i: (0, 0))
    return pl.pallas_call(
        _mlp_body,
        grid=(B // BS,),
        in_specs=[
            bspec_rows(D), bspec_rows(D),
            bspec_rows(1), bspec_rows(1), bspec_rows(1),
            bspec_rows(1), bspec_rows(1), bspec_rows(1),
            bspec_full(ND, D), bspec_full(NM, D), bspec_full(NH, D),
            bspec_full(2, 1), bspec_full(1, 1),
            bspec_full(5 * D + 1, H1), bspec_full(1, H1),
            bspec_full(H1, H2), bspec_full(1, H2),
            bspec_full(H2, H3), bspec_full(1, H3),
        ],
        out_specs=bspec_rows(H3),
        out_shape=jax.ShapeDtypeStruct((B, H3), jnp.float32),
        compiler_params=pltpu.CompilerParams(
            dimension_semantics=("arbitrary",)),
        interpret=interpret,
    )


_mlp = _make_mlp()


def kernel(product_id, user_id, day_of_week, month, hour, rate,
           product_table, user_table, dw_table, month_table, hour_table,
           W_wide, b_wide, W1, b1, W2, b2, W3, b3):
    pid32 = product_id.astype(jnp.int32)
    uid32 = user_id.astype(jnp.int32)
    pe, ue = _make_gather()(pid32, uid32, product_table, user_table)
    pidf = pid32.astype(jnp.float32).reshape(B, 1)
    uidf = uid32.astype(jnp.float32).reshape(B, 1)
    return _mlp(pe, ue,
                day_of_week.astype(jnp.int32).reshape(B, 1),
                month.astype(jnp.int32).reshape(B, 1),
                hour.astype(jnp.int32).reshape(B, 1),
                rate, pidf, uidf,
                dw_table, month_table, hour_table,
                W_wide, b_wide.reshape(1, 1),
                W1, b1.reshape(1, H1), W2, b2.reshape(1, H2),
                W3, b3.reshape(1, H3))


# speedup vs baseline: 2.8281x; 1.0010x over previous
import functools

import jax
import jax.numpy as jnp
from jax import lax
from jax.experimental import pallas as pl
from jax.experimental.pallas import tpu as pltpu
from jax.experimental.pallas import tpu_sc as plsc

B = 4096
D = 40
NC, NS = 2, 16
NW = NC * NS
BPW = B // NW

H1, H2, H3 = 1024, 512, 256
ND, NM, NH = 7, 12, 24
BS = 512


def _gather_body(pid, uid, ptab, utab, pe, ue, idx_v, pe_v, ue_v, sem):
    wid = lax.axis_index("s") * NC + lax.axis_index("c")
    base = wid * BPW
    handles = []

    def fire(idx_hbm, tab, row_buf):
        pltpu.sync_copy(idx_hbm.at[pl.ds(base, BPW)], idx_v)
        for c in range(BPW // 16):
            chunk = idx_v[pl.ds(c * 16, 16)]
            for l in range(16):
                handles.append(pltpu.async_copy(
                    tab.at[pl.ds(chunk[l], 1), :],
                    row_buf.at[pl.ds(c * 16 + l, 1), :], sem))

    fire(pid, ptab, pe_v)
    fire(uid, utab, ue_v)
    for h in handles:
        h.wait()
    pltpu.sync_copy(pe_v, pe.at[pl.ds(base, BPW)])
    pltpu.sync_copy(ue_v, ue.at[pl.ds(base, BPW)])


@functools.cache
def _make_gather():
    return pl.kernel(
        _gather_body,
        out_type=[jax.ShapeDtypeStruct((B, D), jnp.float32)] * 2,
        mesh=plsc.VectorSubcoreMesh(core_axis_name="c", subcore_axis_name="s"),
        scratch_types=[
            pltpu.VMEM((BPW,), jnp.int32),
            pltpu.VMEM((BPW, D), jnp.float32),
            pltpu.VMEM((BPW, D), jnp.float32),
            pltpu.SemaphoreType.DMA,
        ],
    )


def _mlp_body(pe, ue, dow, mon, hr, rate, pidf, uidf,
              dwtab, motab, hrtab, wwide, bwide,
              w1, b1, w2, b2, w3, b3, out):
    def onehot_rows(ids, n, tab):
        oh = (lax.broadcasted_iota(jnp.int32, (BS, n), 1)
              == ids).astype(jnp.float32)
        return jnp.dot(oh, tab[...], precision=lax.Precision.HIGHEST,
                       preferred_element_type=jnp.float32)

    de = onehot_rows(dow[...], ND, dwtab)
    me = onehot_rows(mon[...], NM, motab)
    he = onehot_rows(hr[...], NH, hrtab)
    h = (jnp.dot(pe[...], w1[0:D, :], preferred_element_type=jnp.float32)
         + jnp.dot(ue[...], w1[D:2 * D, :], preferred_element_type=jnp.float32)
         + jnp.dot(de, w1[2 * D:3 * D, :], preferred_element_type=jnp.float32)
         + jnp.dot(me, w1[3 * D:4 * D, :], preferred_element_type=jnp.float32)
         + jnp.dot(he, w1[4 * D:5 * D, :], preferred_element_type=jnp.float32)
         + rate[...] * w1[5 * D:5 * D + 1, :]
         + b1[...])
    h = jnp.maximum(h, 0.0)
    h = jnp.maximum(jnp.dot(h, w2[...], preferred_element_type=jnp.float32)
                    + b2[...], 0.0)
    logits = jnp.dot(h, w3[...], preferred_element_type=jnp.float32) + b3[...]
    m = jnp.max(logits, axis=-1, keepdims=True)
    e = jnp.exp(logits - m)
    sm = e / jnp.sum(e, axis=-1, keepdims=True)
    bf = lambda v: v.astype(jnp.bfloat16).astype(jnp.float32)
    wide = (bf(pidf[...]) * bf(wwide[0, 0]) + bf(uidf[...]) * bf(wwide[1, 0])
            + bwide[0, 0])
    z = sm + wide
    out[...] = 1.0 / (1.0 + jnp.exp(-z))


def _make_mlp(interpret=False):
    bspec_rows = lambda w: pl.BlockSpec((BS, w), lambda i: (i, 0))
    bspec_full = lambda r, c: pl.BlockSpec((r, c), lambda i: (0, 0))
    return pl.pallas_call(
        _mlp_body,
        grid=(B // BS,),
        in_specs=[
            bspec_rows(D), bspec_rows(D),
            bspec_rows(1), bspec_rows(1), bspec_rows(1),
            bspec_rows(1), bspec_rows(1), bspec_rows(1),
            bspec_full(ND, D), bspec_full(NM, D), bspec_full(NH, D),
            bspec_full(2, 1), bspec_full(1, 1),
            bspec_full(5 * D + 1, H1), bspec_full(1, H1),
            bspec_full(H1, H2), bspec_full(1, H2),
            bspec_full(H2, H3), bspec_full(1, H3),
        ],
        out_specs=bspec_rows(H3),
        out_shape=jax.ShapeDtypeStruct((B, H3), jnp.float32),
        compiler_params=pltpu.CompilerParams(
            dimension_semantics=("arbitrary",)),
        interpret=interpret,
    )


_mlp = _make_mlp()


def kernel(product_id, user_id, day_of_week, month, hour, rate,
           product_table, user_table, dw_table, month_table, hour_table,
           W_wide, b_wide, W1, b1, W2, b2, W3, b3):
    pid32 = product_id.astype(jnp.int32)
    uid32 = user_id.astype(jnp.int32)
    pe, ue = _make_gather()(pid32, uid32, product_table, user_table)
    pidf = pid32.astype(jnp.float32).reshape(B, 1)
    uidf = uid32.astype(jnp.float32).reshape(B, 1)
    return _mlp(pe, ue,
                day_of_week.astype(jnp.int32).reshape(B, 1),
                month.astype(jnp.int32).reshape(B, 1),
                hour.astype(jnp.int32).reshape(B, 1),
                rate, pidf, uidf,
                dw_table, month_table, hour_table,
                W_wide, b_wide.reshape(1, 1),
                W1, b1.reshape(1, H1), W2, b2.reshape(1, H2),
                W3, b3.reshape(1, H3))


# speedup vs baseline: 13.8426x; 4.8947x over previous
import functools

import jax
import jax.numpy as jnp
from jax import lax
from jax.experimental import pallas as pl
from jax.experimental.pallas import tpu as pltpu
from jax.experimental.pallas import tpu_sc as plsc

B = 4096
D = 40
NC, NS = 2, 16
NW = NC * NS
BPW = B // NW

H1, H2, H3 = 1024, 512, 256
ND, NM, NH = 7, 12, 24
BS = 512


def _gather_body(pid, uid, ptab_t, utab_t, pe, ue, idx_v, bufs, out_v, sem):
    wid = lax.axis_index("s") * NC + lax.axis_index("c")
    base = wid * BPW
    rows0 = lax.iota(jnp.int32, 16)
    zeros = jnp.zeros((16,), jnp.int32)

    def one_table(tab_t, idx_hbm, out):
        pltpu.sync_copy(idx_hbm.at[pl.ds(base, BPW)], idx_v)

        def chunk_body(c, _):
            chunk = idx_v[pl.ds(pl.multiple_of(c * 16, 16), 16)]
            handles = []
            for l in range(16):
                gt = pl.multiple_of((chunk[l] >> 7) * 128, 128)
                handles.append(pltpu.async_copy(
                    tab_t.at[:, pl.ds(gt, 128)], bufs.at[l], sem))
            for h in handles:
                h.wait()
            for l in range(16):
                lane = zeros + (chunk[l] & 127)
                j = c * 16 + l
                out_v[j, pl.ds(0, 16)] = plsc.load_gather(
                    bufs.at[l], [rows0, lane])
                out_v[j, pl.ds(16, 16)] = plsc.load_gather(
                    bufs.at[l], [rows0 + 16, lane])
                out_v[j, pl.ds(24, 16)] = plsc.load_gather(
                    bufs.at[l], [rows0 + 24, lane])
            return ()

        lax.fori_loop(0, BPW // 16, chunk_body, (), unroll=False)
        pltpu.sync_copy(out_v, out.at[pl.ds(base, BPW)])

    one_table(ptab_t, pid, pe)
    one_table(utab_t, uid, ue)


@functools.cache
def _make_gather():
    return pl.kernel(
        _gather_body,
        out_type=[jax.ShapeDtypeStruct((B, D), jnp.float32)] * 2,
        mesh=plsc.VectorSubcoreMesh(core_axis_name="c", subcore_axis_name="s"),
        scratch_types=[
            pltpu.VMEM((BPW,), jnp.int32),
            pltpu.VMEM((16, D, 128), jnp.float32),
            pltpu.VMEM((BPW, D), jnp.float32),
            pltpu.SemaphoreType.DMA,
        ],
        compiler_params=pltpu.CompilerParams(needs_layout_passes=False),
    )


def _mlp_body(pe, ue, dow, mon, hr, rate, pidf, uidf,
              dwtab, motab, hrtab, wwide, bwide,
              w1, b1, w2, b2, w3, b3, out):
    def onehot_rows(ids, n, tab):
        oh = (lax.broadcasted_iota(jnp.int32, (BS, n), 1)
              == ids).astype(jnp.float32)
        return jnp.dot(oh, tab[...], precision=lax.Precision.HIGHEST,
                       preferred_element_type=jnp.float32)

    de = onehot_rows(dow[...], ND, dwtab)
    me = onehot_rows(mon[...], NM, motab)
    he = onehot_rows(hr[...], NH, hrtab)
    h = (jnp.dot(pe[...], w1[0:D, :], preferred_element_type=jnp.float32)
         + jnp.dot(ue[...], w1[D:2 * D, :], preferred_element_type=jnp.float32)
         + jnp.dot(de, w1[2 * D:3 * D, :], preferred_element_type=jnp.float32)
         + jnp.dot(me, w1[3 * D:4 * D, :], preferred_element_type=jnp.float32)
         + jnp.dot(he, w1[4 * D:5 * D, :], preferred_element_type=jnp.float32)
         + rate[...] * w1[5 * D:5 * D + 1, :]
         + b1[...])
    h = jnp.maximum(h, 0.0)
    h = jnp.maximum(jnp.dot(h, w2[...], preferred_element_type=jnp.float32)
                    + b2[...], 0.0)
    logits = jnp.dot(h, w3[...], preferred_element_type=jnp.float32) + b3[...]
    m = jnp.max(logits, axis=-1, keepdims=True)
    e = jnp.exp(logits - m)
    sm = e / jnp.sum(e, axis=-1, keepdims=True)
    bf = lambda v: v.astype(jnp.bfloat16).astype(jnp.float32)
    wide = (bf(pidf[...]) * bf(wwide[0, 0]) + bf(uidf[...]) * bf(wwide[1, 0])
            + bwide[0, 0])
    z = sm + wide
    out[...] = 1.0 / (1.0 + jnp.exp(-z))


def _make_mlp(interpret=False):
    bspec_rows = lambda w: pl.BlockSpec((BS, w), lambda i: (i, 0))
    bspec_full = lambda r, c: pl.BlockSpec((r, c), lambda i: (0, 0))
    return pl.pallas_call(
        _mlp_body,
        grid=(B // BS,),
        in_specs=[
            bspec_rows(D), bspec_rows(D),
            bspec_rows(1), bspec_rows(1), bspec_rows(1),
            bspec_rows(1), bspec_rows(1), bspec_rows(1),
            bspec_full(ND, D), bspec_full(NM, D), bspec_full(NH, D),
            bspec_full(2, 1), bspec_full(1, 1),
            bspec_full(5 * D + 1, H1), bspec_full(1, H1),
            bspec_full(H1, H2), bspec_full(1, H2),
            bspec_full(H2, H3), bspec_full(1, H3),
        ],
        out_specs=bspec_rows(H3),
        out_shape=jax.ShapeDtypeStruct((B, H3), jnp.float32),
        compiler_params=pltpu.CompilerParams(
            dimension_semantics=("arbitrary",)),
        interpret=interpret,
    )


_mlp = _make_mlp()


def kernel(product_id, user_id, day_of_week, month, hour, rate,
           product_table, user_table, dw_table, month_table, hour_table,
           W_wide, b_wide, W1, b1, W2, b2, W3, b3):
    pid32 = product_id.astype(jnp.int32)
    uid32 = user_id.astype(jnp.int32)
    pe, ue = _make_gather()(pid32, uid32, product_table.T, user_table.T)
    pidf = pid32.astype(jnp.float32).reshape(B, 1)
    uidf = uid32.astype(jnp.float32).reshape(B, 1)
    return _mlp(pe, ue,
                day_of_week.astype(jnp.int32).reshape(B, 1),
                month.astype(jnp.int32).reshape(B, 1),
                hour.astype(jnp.int32).reshape(B, 1),
                rate, pidf, uidf,
                dw_table, month_table, hour_table,
                W_wide, b_wide.reshape(1, 1),
                W1, b1.reshape(1, H1), W2, b2.reshape(1, H2),
                W3, b3.reshape(1, H3))


# speedup vs baseline: 13.8900x; 1.0034x over previous
import functools

import jax
import jax.numpy as jnp
from jax import lax
from jax.experimental import pallas as pl
from jax.experimental.pallas import tpu as pltpu
from jax.experimental.pallas import tpu_sc as plsc

B = 4096
D = 40
NC, NS = 2, 16
NW = NC * NS
BPW = B // NW

H1, H2, H3 = 1024, 512, 256
ND, NM, NH = 7, 12, 24
BS = 512


def _gather_body(pid, uid, ptab_t, utab_t, pe, ue, idx_v, bufs, out_v, sem):
    wid = lax.axis_index("s") * NC + lax.axis_index("c")
    base = wid * BPW
    rows0 = lax.iota(jnp.int32, 16)
    zeros = jnp.zeros((16,), jnp.int32)

    def one_table(tab_t, idx_hbm, out):
        pltpu.sync_copy(idx_hbm.at[pl.ds(base, BPW)], idx_v)

        def chunk_body(c, _):
            chunk = idx_v[pl.ds(pl.multiple_of(c * 16, 16), 16)]
            handles = []
            for l in range(16):
                gt = pl.multiple_of((chunk[l] >> 7) * 128, 128)
                handles.append(pltpu.async_copy(
                    tab_t.at[:, pl.ds(gt, 128)], bufs.at[l], sem))
            for h in handles:
                h.wait()
            for l in range(16):
                lane = zeros + (chunk[l] & 127)
                j = c * 16 + l
                out_v[j, pl.ds(0, 16)] = plsc.load_gather(
                    bufs.at[l], [rows0, lane])
                out_v[j, pl.ds(16, 16)] = plsc.load_gather(
                    bufs.at[l], [rows0 + 16, lane])
                out_v[j, pl.ds(24, 16)] = plsc.load_gather(
                    bufs.at[l], [rows0 + 24, lane])
            return ()

        lax.fori_loop(0, BPW // 16, chunk_body, (), unroll=False)
        pltpu.sync_copy(out_v, out.at[pl.ds(base, BPW)])

    one_table(ptab_t, pid, pe)
    one_table(utab_t, uid, ue)


@functools.cache
def _make_gather():
    return pl.kernel(
        _gather_body,
        out_type=[jax.ShapeDtypeStruct((B, D), jnp.float32)] * 2,
        mesh=plsc.VectorSubcoreMesh(core_axis_name="c", subcore_axis_name="s"),
        scratch_types=[
            pltpu.VMEM((BPW,), jnp.int32),
            pltpu.VMEM((16, D, 128), jnp.float32),
            pltpu.VMEM((BPW, D), jnp.float32),
            pltpu.SemaphoreType.DMA,
        ],
        compiler_params=pltpu.CompilerParams(needs_layout_passes=False),
    )


def _mlp_body(pe, ue, dow, mon, hr, rate, pidf, uidf,
              dwtab, motab, hrtab, wwide, bwide,
              w1, b1, w2, b2, w3, b3, out):
    def onehot_rows(ids, n, tab):
        oh = (lax.broadcasted_iota(jnp.int32, (BS, n), 1)
              == ids).astype(jnp.float32)
        return jnp.dot(oh, tab[...], precision=lax.Precision.HIGHEST,
                       preferred_element_type=jnp.float32)

    de = onehot_rows(dow[...], ND, dwtab)
    me = onehot_rows(mon[...], NM, motab)
    he = onehot_rows(hr[...], NH, hrtab)
    h = (jnp.dot(pe[...], w1[0:D, :], preferred_element_type=jnp.float32)
         + jnp.dot(ue[...], w1[D:2 * D, :], preferred_element_type=jnp.float32)
         + jnp.dot(de, w1[2 * D:3 * D, :], preferred_element_type=jnp.float32)
         + jnp.dot(me, w1[3 * D:4 * D, :], preferred_element_type=jnp.float32)
         + jnp.dot(he, w1[4 * D:5 * D, :], preferred_element_type=jnp.float32)
         + rate[...] * w1[5 * D:5 * D + 1, :]
         + b1[...])
    h = jnp.maximum(h, 0.0)
    h = jnp.maximum(jnp.dot(h, w2[...], preferred_element_type=jnp.float32)
                    + b2[...], 0.0)
    logits = jnp.dot(h, w3[...], preferred_element_type=jnp.float32) + b3[...]
    m = jnp.max(logits, axis=-1, keepdims=True)
    e = jnp.exp(logits - m)
    sm = e / jnp.sum(e, axis=-1, keepdims=True)
    bf = lambda v: v.astype(jnp.bfloat16).astype(jnp.float32)
    wide = (bf(pidf[...]) * bf(wwide[0, 0]) + bf(uidf[...]) * bf(wwide[1, 0])
            + bwide[0, 0])
    z = sm + wide
    out[...] = 1.0 / (1.0 + jnp.exp(-z))


def _make_mlp():
    bspec_rows = lambda w: pl.BlockSpec((BS, w), lambda i: (i, 0))
    bspec_full = lambda r, c: pl.BlockSpec((r, c), lambda i: (0, 0))
    return pl.pallas_call(
        _mlp_body,
        grid=(B // BS,),
        in_specs=[
            bspec_rows(D), bspec_rows(D),
            bspec_rows(1), bspec_rows(1), bspec_rows(1),
            bspec_rows(1), bspec_rows(1), bspec_rows(1),
            bspec_full(ND, D), bspec_full(NM, D), bspec_full(NH, D),
            bspec_full(2, 1), bspec_full(1, 1),
            bspec_full(5 * D + 1, H1), bspec_full(1, H1),
            bspec_full(H1, H2), bspec_full(1, H2),
            bspec_full(H2, H3), bspec_full(1, H3),
        ],
        out_specs=bspec_rows(H3),
        out_shape=jax.ShapeDtypeStruct((B, H3), jnp.float32),
        compiler_params=pltpu.CompilerParams(
            dimension_semantics=("arbitrary",)),
    )


_mlp = _make_mlp()


def kernel(product_id, user_id, day_of_week, month, hour, rate,
           product_table, user_table, dw_table, month_table, hour_table,
           W_wide, b_wide, W1, b1, W2, b2, W3, b3):
    pid32 = product_id.astype(jnp.int32)
    uid32 = user_id.astype(jnp.int32)
    pe, ue = _make_gather()(pid32, uid32, product_table.T, user_table.T)
    pidf = pid32.astype(jnp.float32).reshape(B, 1)
    uidf = uid32.astype(jnp.float32).reshape(B, 1)
    return _mlp(pe, ue,
                day_of_week.astype(jnp.int32).reshape(B, 1),
                month.astype(jnp.int32).reshape(B, 1),
                hour.astype(jnp.int32).reshape(B, 1),
                rate, pidf, uidf,
                dw_table, month_table, hour_table,
                W_wide, b_wide.reshape(1, 1),
                W1, b1.reshape(1, H1), W2, b2.reshape(1, H2),
                W3, b3.reshape(1, H3))
